# baseline (speedup 1.0000x reference)
import functools

import jax
import jax.numpy as jnp
from jax import lax
from jax.experimental import pallas as pl
from jax.experimental.pallas import tpu as pltpu
from jax.experimental.pallas import tpu_sc as plsc

E = 8
TOP_K = 2
D = 2048
DW = D // 2
F = 1408
BG = 256
SB = 512
NC = 2
NS = 16
NW = NC * NS
CH = 32


def _gate_kernel(x_ref, gw_ref, e_ref, w_ref):
    x = x_ref[...]
    logits = jax.lax.dot_general(
        x, gw_ref[...], (((1,), (1,)), ((), ())),
        preferred_element_type=jnp.float32)
    tb = logits.shape[0]
    lane = jax.lax.broadcasted_iota(jnp.int32, (tb, E), 1)
    m1 = jnp.max(logits, axis=1, keepdims=True)
    e1 = jnp.min(jnp.where(logits == m1, lane, E), axis=1, keepdims=True)
    masked = jnp.where(lane == e1, -jnp.inf, logits)
    m2 = jnp.max(masked, axis=1, keepdims=True)
    e2 = jnp.min(jnp.where(masked == m2, lane, E), axis=1, keepdims=True)
    w1 = 1.0 / (1.0 + jnp.exp(m2 - m1))
    e_ref[...] = jnp.concatenate([e1, e2], axis=1)
    w_ref[...] = jnp.concatenate([w1, 1.0 - w1], axis=1)


def _sort_kernel(e_ref, dest_ref, gid_ref, cnt_ref, car_ref, off_ref):
    p = pl.program_id(0)
    b = pl.program_id(1)

    @pl.when((p == 0) & (b == 0))
    def _():
        cnt_ref[...] = jnp.zeros_like(cnt_ref)
        gid_ref[...] = jnp.zeros_like(gid_ref)

    ev = e_ref[0, 0, :].astype(jnp.float32)
    sub = jax.lax.broadcasted_iota(jnp.int32, (E, SB), 0).astype(jnp.float32)
    oh = (jnp.broadcast_to(ev[None, :], (E, SB)) == sub).astype(jnp.float32)
    counts = jnp.sum(oh, axis=1, keepdims=True)

    @pl.when(p == 0)
    def _():
        cnt_ref[:, :1] += counts

    @pl.when((p == 1) & (b == 0))
    def _():
        tot = cnt_ref[:, :1]
        ac = jnp.floor((tot + (BG - 1)) / BG) * BG
        tril = (jax.lax.broadcasted_iota(jnp.int32, (E, E), 1)
                <= jax.lax.broadcasted_iota(jnp.int32, (E, E), 0)
                ).astype(jnp.float32)
        end = jnp.dot(tril, ac, preferred_element_type=jnp.float32)
        off_ref[:, :1] = end - ac
        car_ref[...] = jnp.zeros_like(car_ref)
        g = jax.lax.broadcasted_iota(jnp.int32, (E, 128), 1).astype(
            jnp.float32) * BG
        gid = jnp.sum((jnp.broadcast_to(end, (E, 128)) <= g).astype(
            jnp.float32), axis=0, keepdims=True)
        gid_ref[...] = jnp.minimum(gid, E - 1).astype(jnp.int32)

    @pl.when(p == 1)
    def _():
        s_i = jax.lax.broadcasted_iota(jnp.int32, (SB, SB), 0)
        s_j = jax.lax.broadcasted_iota(jnp.int32, (SB, SB), 1)
        ustrict = (s_i < s_j).astype(jnp.float32)
        pref = jnp.dot(oh, ustrict, preferred_element_type=jnp.float32)
        base = off_ref[:, :1] + car_ref[:, :1]
        dest = jnp.sum(oh * (pref + jnp.broadcast_to(base, (E, SB))),
                       axis=0, keepdims=True)
        dest_ref[...] = dest.astype(jnp.int32)[None]
        car_ref[:, :1] += counts


def _group_mm_kernel(gid_ref, xs_ref, gw_ref, uw_ref, dw_ref, ys_ref):
    x = xs_ref[...]
    g = jnp.dot(x, gw_ref[0], preferred_element_type=jnp.float32)
    u = jnp.dot(x, uw_ref[0], preferred_element_type=jnp.float32)
    h = (g * jax.lax.logistic(g) * u).astype(jnp.bfloat16)
    ys_ref[...] = jnp.dot(h, dw_ref[0],
                          preferred_element_type=jnp.float32).astype(
                              jnp.bfloat16)


def _final_kernel(x_ref, yg_ref, wt_ref, gw_ref, uw_ref, dw_ref, o_ref):
    x = x_ref[...].astype(jnp.bfloat16)
    g = jnp.dot(x, gw_ref[...], preferred_element_type=jnp.float32)
    u = jnp.dot(x, uw_ref[...], preferred_element_type=jnp.float32)
    h = (g * jax.lax.logistic(g) * u).astype(jnp.bfloat16)
    z = jnp.dot(h, dw_ref[...], preferred_element_type=jnp.float32)
    yg = yg_ref[...].astype(jnp.float32)
    w0 = wt_ref[:, :1]
    w1 = wt_ref[:, 1:]
    o_ref[...] = z + w0 * yg[:, :D] + w1 * yg[:, D:]


def _make_dispatch_kernel(s, s_pad, t):
    chunks = s // CH // NW
    mesh = plsc.VectorSubcoreMesh(core_axis_name="c", subcore_axis_name="s")

    @functools.partial(
        pl.kernel, mesh=mesh,
        out_type=jax.ShapeDtypeStruct((s_pad, DW), jnp.int32),
        scratch_types=[
            pltpu.VMEM((chunks, CH), jnp.int32),
            pltpu.VMEM((chunks, CH), jnp.int32),
            pltpu.VMEM((CH, DW), jnp.int32),
            pltpu.VMEM((CH, DW), jnp.int32),
            pltpu.SemaphoreType.DMA,
            pltpu.SemaphoreType.DMA,
        ],
    )
    def dispatch_kernel(tok_hbm, dest_hbm, x_hbm, xs_hbm,
                        tok_v, dst_v, buf0, buf1, sem_g, sem_s):
        wid = lax.axis_index("s") * NC + lax.axis_index("c")
        pltpu.sync_copy(tok_hbm.at[wid], tok_v)
        pltpu.sync_copy(dest_hbm.at[wid], dst_v)
        bufs = (buf0, buf1)
        gets = {}
        puts = {}
        gets[0] = pltpu.async_copy(x_hbm.at[tok_v.at[0]], bufs[0], sem_g)
        for j in range(chunks):
            b = bufs[j % 2]
            gets[j].wait()
            if j >= 1:
                puts[j - 1].wait()
            if j + 1 < chunks:
                gets[j + 1] = pltpu.async_copy(
                    x_hbm.at[tok_v.at[j + 1]], bufs[(j + 1) % 2], sem_g)
            puts[j] = pltpu.async_copy(b, xs_hbm.at[dst_v.at[j]], sem_s)
        puts[chunks - 1].wait()

    return dispatch_kernel


def _make_combine_kernel(s, s_pad):
    chunks = s // CH // NW
    mesh = plsc.VectorSubcoreMesh(core_axis_name="c", subcore_axis_name="s")

    @functools.partial(
        pl.kernel, mesh=mesh,
        out_type=jax.ShapeDtypeStruct((s, DW), jnp.int32),
        scratch_types=[
            pltpu.VMEM((chunks, CH), jnp.int32),
            pltpu.VMEM((CH, DW), jnp.int32),
            pltpu.VMEM((CH, DW), jnp.int32),
            pltpu.SemaphoreType.DMA,
            pltpu.SemaphoreType.DMA,
        ],
    )
    def combine_kernel(dest_hbm, ys_hbm, out_hbm,
                       dst_v, buf0, buf1, sem_g, sem_s):
        wid = lax.axis_index("s") * NC + lax.axis_index("c")
        r0 = wid * chunks * CH
        pltpu.sync_copy(dest_hbm.at[wid], dst_v)
        bufs = (buf0, buf1)
        gets = {}
        puts = {}
        gets[0] = pltpu.async_copy(ys_hbm.at[dst_v.at[0]], bufs[0], sem_g)
        for j in range(chunks):
            b = bufs[j % 2]
            gets[j].wait()
            if j >= 1:
                puts[j - 1].wait()
            if j + 1 < chunks:
                gets[j + 1] = pltpu.async_copy(
                    ys_hbm.at[dst_v.at[j + 1]], bufs[(j + 1) % 2], sem_g)
            puts[j] = pltpu.async_copy(
                b, out_hbm.at[pl.ds(r0 + j * CH, CH)], sem_s)
        puts[chunks - 1].wait()

    return combine_kernel


@jax.jit
def kernel(hidden_states, gate_weight, expert_gate_w, expert_up_w,
           expert_down_w, shared_gate_w, shared_up_w, shared_down_w):
    orig_shape = hidden_states.shape
    x = hidden_states.reshape(-1, D)
    t = x.shape[0]
    s = t * TOP_K
    s_pad = s + E * BG
    nblk = s // SB
    tb = min(t, 512)

    topk_e, topk_w = pl.pallas_call(
        _gate_kernel,
        grid=(t // tb,),
        in_specs=[
            pl.BlockSpec((tb, D), lambda i: (i, 0)),
            pl.BlockSpec((E, D), lambda i: (0, 0)),
        ],
        out_specs=[
            pl.BlockSpec((tb, TOP_K), lambda i: (i, 0)),
            pl.BlockSpec((tb, TOP_K), lambda i: (i, 0)),
        ],
        out_shape=[
            jax.ShapeDtypeStruct((t, TOP_K), jnp.int32),
            jax.ShapeDtypeStruct((t, TOP_K), jnp.float32),
        ],
    )(x, gate_weight)

    eflat3d = topk_e.reshape(nblk, 1, SB)
    dest3d, gid128 = pl.pallas_call(
        _sort_kernel,
        grid=(2, nblk),
        in_specs=[pl.BlockSpec((1, 1, SB), lambda p, b: (b, 0, 0))],
        out_specs=[
            pl.BlockSpec((1, 1, SB), lambda p, b: (b, 0, 0)),
            pl.BlockSpec((1, 128), lambda p, b: (0, 0)),
        ],
        out_shape=[
            jax.ShapeDtypeStruct((nblk, 1, SB), jnp.int32),
            jax.ShapeDtypeStruct((1, 128), jnp.int32),
        ],
        scratch_shapes=[
            pltpu.VMEM((E, 128), jnp.float32),
            pltpu.VMEM((E, 128), jnp.float32),
            pltpu.VMEM((E, 128), jnp.float32),
        ],
    )(eflat3d)
    gid = gid128.reshape(128)[:s_pad // BG]

    bf = jnp.bfloat16
    tokidx = (jnp.arange(s, dtype=jnp.int32) // TOP_K).reshape(
        NW, s // CH // NW, CH)
    dest_t = dest3d.reshape(NW, s // CH // NW, CH)

    x_i32 = lax.bitcast_convert_type(
        x.astype(bf).reshape(t, DW, 2), jnp.int32)
    xs_i32 = _make_dispatch_kernel(s, s_pad, t)(tokidx, dest_t, x_i32)
    xs = lax.bitcast_convert_type(xs_i32, bf).reshape(s_pad, D)

    ys = pl.pallas_call(
        _group_mm_kernel,
        grid_spec=pltpu.PrefetchScalarGridSpec(
            num_scalar_prefetch=1,
            grid=(s_pad // BG,),
            in_specs=[
                pl.BlockSpec((BG, D), lambda g, gid_r: (g, 0)),
                pl.BlockSpec((1, D, F), lambda g, gid_r: (gid_r[g], 0, 0)),
                pl.BlockSpec((1, D, F), lambda g, gid_r: (gid_r[g], 0, 0)),
                pl.BlockSpec((1, F, D), lambda g, gid_r: (gid_r[g], 0, 0)),
            ],
            out_specs=pl.BlockSpec((BG, D), lambda g, gid_r: (g, 0)),
        ),
        out_shape=jax.ShapeDtypeStruct((s_pad, D), bf),
        compiler_params=pltpu.CompilerParams(
            dimension_semantics=("arbitrary",)),
    )(gid, xs, expert_gate_w.astype(bf), expert_up_w.astype(bf),
      expert_down_w.astype(bf))

    ys_i32 = lax.bitcast_convert_type(ys.reshape(s_pad, DW, 2), jnp.int32)
    ygf_i32 = _make_combine_kernel(s, s_pad)(dest_t, ys_i32)
    yg = lax.bitcast_convert_type(ygf_i32, bf).reshape(t, TOP_K * D)

    tb2 = min(t, 256)
    y = pl.pallas_call(
        _final_kernel,
        grid=(t // tb2,),
        in_specs=[
            pl.BlockSpec((tb2, D), lambda i: (i, 0)),
            pl.BlockSpec((tb2, TOP_K * D), lambda i: (i, 0)),
            pl.BlockSpec((tb2, TOP_K), lambda i: (i, 0)),
            pl.BlockSpec((D, TOP_K * F), lambda i: (0, 0)),
            pl.BlockSpec((D, TOP_K * F), lambda i: (0, 0)),
            pl.BlockSpec((TOP_K * F, D), lambda i: (0, 0)),
        ],
        out_specs=pl.BlockSpec((tb2, D), lambda i: (i, 0)),
        out_shape=jax.ShapeDtypeStruct((t, D), jnp.float32),
    )(x, yg, topk_w, shared_gate_w.astype(bf), shared_up_w.astype(bf),
      shared_down_w.astype(bf))

    return y.reshape(orig_shape)


# speedup vs baseline: 17.5046x; 17.5046x over previous
import functools

import jax
import jax.numpy as jnp
from jax import lax
from jax.experimental import pallas as pl
from jax.experimental.pallas import tpu as pltpu
from jax.experimental.pallas import tpu_sc as plsc

E = 8
TOP_K = 2
D = 2048
DW = D
F = 1408
BG = 256
SB = 512
NC = 2
NS = 16
NW = NC * NS
CH = 16


def _gate_kernel(x_ref, gw_ref, e_ref, w_ref):
    x = x_ref[...]
    logits = jax.lax.dot_general(
        x, gw_ref[...], (((1,), (1,)), ((), ())),
        preferred_element_type=jnp.float32)
    tb = logits.shape[0]
    lane = jax.lax.broadcasted_iota(jnp.int32, (tb, E), 1)
    m1 = jnp.max(logits, axis=1, keepdims=True)
    e1 = jnp.min(jnp.where(logits == m1, lane, E), axis=1, keepdims=True)
    masked = jnp.where(lane == e1, -jnp.inf, logits)
    m2 = jnp.max(masked, axis=1, keepdims=True)
    e2 = jnp.min(jnp.where(masked == m2, lane, E), axis=1, keepdims=True)
    w1 = 1.0 / (1.0 + jnp.exp(m2 - m1))
    e_ref[...] = jnp.concatenate([e1, e2], axis=1)
    w_ref[...] = jnp.concatenate([w1, 1.0 - w1], axis=1)


def _sort_kernel(e_ref, dest_ref, gid_ref, cnt_ref, car_ref, off_ref):
    p = pl.program_id(0)
    b = pl.program_id(1)

    @pl.when((p == 0) & (b == 0))
    def _():
        cnt_ref[...] = jnp.zeros_like(cnt_ref)
        gid_ref[...] = jnp.zeros_like(gid_ref)

    ev = e_ref[0, 0, :].astype(jnp.float32)
    sub = jax.lax.broadcasted_iota(jnp.int32, (E, SB), 0).astype(jnp.float32)
    oh = (jnp.broadcast_to(ev[None, :], (E, SB)) == sub).astype(jnp.float32)
    counts = jnp.sum(oh, axis=1, keepdims=True)

    @pl.when(p == 0)
    def _():
        cnt_ref[:, :1] += counts

    @pl.when((p == 1) & (b == 0))
    def _():
        tot = cnt_ref[:, :1]
        ac = jnp.floor((tot + (BG - 1)) / BG) * BG
        tril = (jax.lax.broadcasted_iota(jnp.int32, (E, E), 1)
                <= jax.lax.broadcasted_iota(jnp.int32, (E, E), 0)
                ).astype(jnp.float32)
        end = jnp.dot(tril, ac, preferred_element_type=jnp.float32)
        off_ref[:, :1] = end - ac
        car_ref[...] = jnp.zeros_like(car_ref)
        g = jax.lax.broadcasted_iota(jnp.int32, (E, 128), 1).astype(
            jnp.float32) * BG
        gid = jnp.sum((jnp.broadcast_to(end, (E, 128)) <= g).astype(
            jnp.float32), axis=0, keepdims=True)
        gid_ref[...] = jnp.minimum(gid, E - 1).astype(jnp.int32)

    @pl.when(p == 1)
    def _():
        s_i = jax.lax.broadcasted_iota(jnp.int32, (SB, SB), 0)
        s_j = jax.lax.broadcasted_iota(jnp.int32, (SB, SB), 1)
        ustrict = (s_i < s_j).astype(jnp.float32)
        pref = jnp.dot(oh, ustrict, preferred_element_type=jnp.float32)
        base = off_ref[:, :1] + car_ref[:, :1]
        dest = jnp.sum(oh * (pref + jnp.broadcast_to(base, (E, SB))),
                       axis=0, keepdims=True)
        dest_ref[...] = dest.astype(jnp.int32)[None]
        car_ref[:, :1] += counts


def _group_mm_kernel(gid_ref, xs_ref, gw_ref, uw_ref, dw_ref, ys_ref):
    x = xs_ref[...].astype(jnp.bfloat16)
    g = jnp.dot(x, gw_ref[0], preferred_element_type=jnp.float32)
    u = jnp.dot(x, uw_ref[0], preferred_element_type=jnp.float32)
    h = (g * jax.lax.logistic(g) * u).astype(jnp.bfloat16)
    ys_ref[...] = jnp.dot(h, dw_ref[0], preferred_element_type=jnp.float32)


def _final_kernel(x_ref, yg_ref, wt_ref, gw_ref, uw_ref, dw_ref, o_ref):
    x = x_ref[...].astype(jnp.bfloat16)
    g = jnp.dot(x, gw_ref[...], preferred_element_type=jnp.float32)
    u = jnp.dot(x, uw_ref[...], preferred_element_type=jnp.float32)
    h = (g * jax.lax.logistic(g) * u).astype(jnp.bfloat16)
    z = jnp.dot(h, dw_ref[...], preferred_element_type=jnp.float32)
    yg = yg_ref[...]
    w0 = wt_ref[:, :1]
    w1 = wt_ref[:, 1:]
    o_ref[...] = z + w0 * yg[:, :D] + w1 * yg[:, D:]


def _make_dispatch_kernel(s, s_pad, t):
    chunks = s // CH // NW
    mesh = plsc.VectorSubcoreMesh(core_axis_name="c", subcore_axis_name="s")

    @functools.partial(
        pl.kernel, mesh=mesh,
        out_type=jax.ShapeDtypeStruct((s_pad, DW), jnp.float32),
        scratch_types=[
            pltpu.VMEM((chunks, CH), jnp.int32),
            pltpu.VMEM((chunks, CH), jnp.int32),
            pltpu.VMEM((CH, DW), jnp.float32),
            pltpu.VMEM((CH, DW), jnp.float32),
            pltpu.SemaphoreType.DMA,
            pltpu.SemaphoreType.DMA,
        ],
    )
    def dispatch_kernel(tok_hbm, dest_hbm, x_hbm, xs_hbm,
                        tok_v, dst_v, buf0, buf1, sem_g, sem_s):
        wid = lax.axis_index("s") * NC + lax.axis_index("c")
        pltpu.sync_copy(tok_hbm.at[wid], tok_v)
        pltpu.sync_copy(dest_hbm.at[wid], dst_v)
        bufs = (buf0, buf1)
        gets = {}
        puts = {}
        gets[0] = pltpu.async_copy(x_hbm.at[tok_v.at[0]], bufs[0], sem_g)
        for j in range(chunks):
            b = bufs[j % 2]
            gets[j].wait()
            if j >= 1:
                puts[j - 1].wait()
            if j + 1 < chunks:
                gets[j + 1] = pltpu.async_copy(
                    x_hbm.at[tok_v.at[j + 1]], bufs[(j + 1) % 2], sem_g)
            puts[j] = pltpu.async_copy(b, xs_hbm.at[dst_v.at[j]], sem_s)
        puts[chunks - 1].wait()

    return dispatch_kernel


def _make_combine_kernel(s, s_pad):
    chunks = s // CH // NW
    mesh = plsc.VectorSubcoreMesh(core_axis_name="c", subcore_axis_name="s")

    @functools.partial(
        pl.kernel, mesh=mesh,
        out_type=jax.ShapeDtypeStruct((s, DW), jnp.float32),
        scratch_types=[
            pltpu.VMEM((chunks, CH), jnp.int32),
            pltpu.VMEM((CH, DW), jnp.float32),
            pltpu.VMEM((CH, DW), jnp.float32),
            pltpu.SemaphoreType.DMA,
            pltpu.SemaphoreType.DMA,
        ],
    )
    def combine_kernel(dest_hbm, ys_hbm, out_hbm,
                       dst_v, buf0, buf1, sem_g, sem_s):
        wid = lax.axis_index("s") * NC + lax.axis_index("c")
        r0 = wid * chunks * CH
        pltpu.sync_copy(dest_hbm.at[wid], dst_v)
        bufs = (buf0, buf1)
        gets = {}
        puts = {}
        gets[0] = pltpu.async_copy(ys_hbm.at[dst_v.at[0]], bufs[0], sem_g)
        for j in range(chunks):
            b = bufs[j % 2]
            gets[j].wait()
            if j >= 1:
                puts[j - 1].wait()
            if j + 1 < chunks:
                gets[j + 1] = pltpu.async_copy(
                    ys_hbm.at[dst_v.at[j + 1]], bufs[(j + 1) % 2], sem_g)
            puts[j] = pltpu.async_copy(
                b, out_hbm.at[pl.ds(r0 + j * CH, CH)], sem_s)
        puts[chunks - 1].wait()

    return combine_kernel


@jax.jit
def kernel(hidden_states, gate_weight, expert_gate_w, expert_up_w,
           expert_down_w, shared_gate_w, shared_up_w, shared_down_w):
    orig_shape = hidden_states.shape
    x = hidden_states.reshape(-1, D)
    t = x.shape[0]
    s = t * TOP_K
    s_pad = s + E * BG
    nblk = s // SB
    tb = min(t, 512)

    topk_e, topk_w = pl.pallas_call(
        _gate_kernel,
        grid=(t // tb,),
        in_specs=[
            pl.BlockSpec((tb, D), lambda i: (i, 0)),
            pl.BlockSpec((E, D), lambda i: (0, 0)),
        ],
        out_specs=[
            pl.BlockSpec((tb, TOP_K), lambda i: (i, 0)),
            pl.BlockSpec((tb, TOP_K), lambda i: (i, 0)),
        ],
        out_shape=[
            jax.ShapeDtypeStruct((t, TOP_K), jnp.int32),
            jax.ShapeDtypeStruct((t, TOP_K), jnp.float32),
        ],
    )(x, gate_weight)

    eflat3d = topk_e.reshape(nblk, 1, SB)
    dest3d, gid128 = pl.pallas_call(
        _sort_kernel,
        grid=(2, nblk),
        in_specs=[pl.BlockSpec((1, 1, SB), lambda p, b: (b, 0, 0))],
        out_specs=[
            pl.BlockSpec((1, 1, SB), lambda p, b: (b, 0, 0)),
            pl.BlockSpec((1, 128), lambda p, b: (0, 0)),
        ],
        out_shape=[
            jax.ShapeDtypeStruct((nblk, 1, SB), jnp.int32),
            jax.ShapeDtypeStruct((1, 128), jnp.int32),
        ],
        scratch_shapes=[
            pltpu.VMEM((E, 128), jnp.float32),
            pltpu.VMEM((E, 128), jnp.float32),
            pltpu.VMEM((E, 128), jnp.float32),
        ],
    )(eflat3d)
    gid = gid128.reshape(128)[:s_pad // BG]

    bf = jnp.bfloat16
    tokidx = (jnp.arange(s, dtype=jnp.int32) // TOP_K).reshape(
        NW, s // CH // NW, CH)
    dest_t = dest3d.reshape(NW, s // CH // NW, CH)

    xs = _make_dispatch_kernel(s, s_pad, t)(tokidx, dest_t, x)

    ys = pl.pallas_call(
        _group_mm_kernel,
        grid_spec=pltpu.PrefetchScalarGridSpec(
            num_scalar_prefetch=1,
            grid=(s_pad // BG,),
            in_specs=[
                pl.BlockSpec((BG, D), lambda g, gid_r: (g, 0)),
                pl.BlockSpec((1, D, F), lambda g, gid_r: (gid_r[g], 0, 0)),
                pl.BlockSpec((1, D, F), lambda g, gid_r: (gid_r[g], 0, 0)),
                pl.BlockSpec((1, F, D), lambda g, gid_r: (gid_r[g], 0, 0)),
            ],
            out_specs=pl.BlockSpec((BG, D), lambda g, gid_r: (g, 0)),
        ),
        out_shape=jax.ShapeDtypeStruct((s_pad, D), jnp.float32),
        compiler_params=pltpu.CompilerParams(
            dimension_semantics=("arbitrary",)),
    )(gid, xs, expert_gate_w.astype(bf), expert_up_w.astype(bf),
      expert_down_w.astype(bf))

    ygf = _make_combine_kernel(s, s_pad)(dest_t, ys)
    yg = ygf.reshape(t, TOP_K * D)

    tb2 = min(t, 256)
    y = pl.pallas_call(
        _final_kernel,
        grid=(t // tb2,),
        in_specs=[
            pl.BlockSpec((tb2, D), lambda i: (i, 0)),
            pl.BlockSpec((tb2, TOP_K * D), lambda i: (i, 0)),
            pl.BlockSpec((tb2, TOP_K), lambda i: (i, 0)),
            pl.BlockSpec((D, TOP_K * F), lambda i: (0, 0)),
            pl.BlockSpec((D, TOP_K * F), lambda i: (0, 0)),
            pl.BlockSpec((TOP_K * F, D), lambda i: (0, 0)),
        ],
        out_specs=pl.BlockSpec((tb2, D), lambda i: (i, 0)),
        out_shape=jax.ShapeDtypeStruct((t, D), jnp.float32),
    )(x, yg, topk_w, shared_gate_w.astype(bf), shared_up_w.astype(bf),
      shared_down_w.astype(bf))

    return y.reshape(orig_shape)


# speedup vs baseline: 17.5108x; 1.0004x over previous
import functools

import jax
import jax.numpy as jnp
import numpy as np
from jax import lax
from jax.experimental import pallas as pl
from jax.experimental.pallas import tpu as pltpu
from jax.experimental.pallas import tpu_sc as plsc

E = 8
TOP_K = 2
D = 2048
DW = D
F = 1408
BG = 256
SB = 512
NC = 2
NS = 16
NW = NC * NS
CH = 16


def _gate_kernel(x_ref, gw_ref, e_ref, w_ref):
    x = x_ref[...]
    logits = jax.lax.dot_general(
        x, gw_ref[...], (((1,), (1,)), ((), ())),
        preferred_element_type=jnp.float32)
    tb = logits.shape[0]
    lane = jax.lax.broadcasted_iota(jnp.int32, (tb, E), 1)
    m1 = jnp.max(logits, axis=1, keepdims=True)
    e1 = jnp.min(jnp.where(logits == m1, lane, E), axis=1, keepdims=True)
    masked = jnp.where(lane == e1, -jnp.inf, logits)
    m2 = jnp.max(masked, axis=1, keepdims=True)
    e2 = jnp.min(jnp.where(masked == m2, lane, E), axis=1, keepdims=True)
    w1 = 1.0 / (1.0 + jnp.exp(m2 - m1))
    e_ref[...] = jnp.concatenate([e1, e2], axis=1)
    w_ref[...] = jnp.concatenate([w1, 1.0 - w1], axis=1)


def _sort_kernel(e_ref, tri_ref, dest_ref, gid_ref, cnt_ref, car_ref,
                 off_ref):
    p = pl.program_id(0)
    b = pl.program_id(1)

    @pl.when((p == 0) & (b == 0))
    def _():
        cnt_ref[...] = jnp.zeros_like(cnt_ref)
        gid_ref[...] = jnp.zeros_like(gid_ref)

    ev = e_ref[0, 0, :].astype(jnp.float32)
    sub = jax.lax.broadcasted_iota(jnp.int32, (E, SB), 0).astype(jnp.float32)
    oh = (jnp.broadcast_to(ev[None, :], (E, SB)) == sub).astype(jnp.float32)
    counts = jnp.sum(oh, axis=1, keepdims=True)

    @pl.when(p == 0)
    def _():
        cnt_ref[:, :1] += counts

    @pl.when((p == 1) & (b == 0))
    def _():
        tot = cnt_ref[:, :1]
        ac = jnp.floor((tot + (BG - 1)) / BG) * BG
        tril = (jax.lax.broadcasted_iota(jnp.int32, (E, E), 1)
                <= jax.lax.broadcasted_iota(jnp.int32, (E, E), 0)
                ).astype(jnp.float32)
        end = jnp.dot(tril, ac, preferred_element_type=jnp.float32)
        off_ref[:, :1] = end - ac
        car_ref[...] = jnp.zeros_like(car_ref)
        g = jax.lax.broadcasted_iota(jnp.int32, (E, 128), 1).astype(
            jnp.float32) * BG
        gid = jnp.sum((jnp.broadcast_to(end, (E, 128)) <= g).astype(
            jnp.float32), axis=0, keepdims=True)
        gid_ref[...] = jnp.minimum(gid, E - 1).astype(jnp.int32)

    @pl.when(p == 1)
    def _():
        pref = jnp.dot(oh, tri_ref[...], preferred_element_type=jnp.float32)
        base = off_ref[:, :1] + car_ref[:, :1]
        dest = jnp.sum(oh * (pref + jnp.broadcast_to(base, (E, SB))),
                       axis=0, keepdims=True)
        dest_ref[...] = dest.astype(jnp.int32)[None]
        car_ref[:, :1] += counts


def _group_mm_kernel(gid_ref, xs_ref, gw_ref, uw_ref, dw_ref, ys_ref):
    x = xs_ref[...].astype(jnp.bfloat16)
    g = jnp.dot(x, gw_ref[0], preferred_element_type=jnp.float32)
    u = jnp.dot(x, uw_ref[0], preferred_element_type=jnp.float32)
    h = (g * jax.lax.logistic(g) * u).astype(jnp.bfloat16)
    ys_ref[...] = jnp.dot(h, dw_ref[0], preferred_element_type=jnp.float32)


def _final_kernel(x_ref, yg_ref, wt_ref, gw_ref, uw_ref, dw_ref, o_ref):
    x = x_ref[...].astype(jnp.bfloat16)
    g = jnp.dot(x, gw_ref[...], preferred_element_type=jnp.float32)
    u = jnp.dot(x, uw_ref[...], preferred_element_type=jnp.float32)
    h = (g * jax.lax.logistic(g) * u).astype(jnp.bfloat16)
    z = jnp.dot(h, dw_ref[...], preferred_element_type=jnp.float32)
    yg = yg_ref[...]
    w0 = wt_ref[:, :1]
    w1 = wt_ref[:, 1:]
    o_ref[...] = z + w0 * yg[:, :D] + w1 * yg[:, D:]


def _make_dispatch_kernel(s, s_pad, t):
    chunks = s // CH // NW
    mesh = plsc.VectorSubcoreMesh(core_axis_name="c", subcore_axis_name="s")

    @functools.partial(
        pl.kernel, mesh=mesh,
        out_type=jax.ShapeDtypeStruct((s_pad, DW), jnp.float32),
        scratch_types=[
            pltpu.VMEM((chunks, CH), jnp.int32),
            pltpu.VMEM((chunks, CH), jnp.int32),
            pltpu.VMEM((CH, DW), jnp.float32),
            pltpu.VMEM((CH, DW), jnp.float32),
            pltpu.SemaphoreType.DMA,
            pltpu.SemaphoreType.DMA,
        ],
    )
    def dispatch_kernel(tok_hbm, dest_hbm, x_hbm, xs_hbm,
                        tok_v, dst_v, buf0, buf1, sem_g, sem_s):
        wid = lax.axis_index("s") * NC + lax.axis_index("c")
        pltpu.sync_copy(tok_hbm.at[wid], tok_v)
        pltpu.sync_copy(dest_hbm.at[wid], dst_v)
        bufs = (buf0, buf1)
        gets = {}
        puts = {}
        gets[0] = pltpu.async_copy(x_hbm.at[tok_v.at[0]], bufs[0], sem_g)
        for j in range(chunks):
            b = bufs[j % 2]
            gets[j].wait()
            if j >= 1:
                puts[j - 1].wait()
            if j + 1 < chunks:
                gets[j + 1] = pltpu.async_copy(
                    x_hbm.at[tok_v.at[j + 1]], bufs[(j + 1) % 2], sem_g)
            puts[j] = pltpu.async_copy(b, xs_hbm.at[dst_v.at[j]], sem_s)
        puts[chunks - 1].wait()

    return dispatch_kernel


def _make_combine_kernel(s, s_pad):
    chunks = s // CH // NW
    mesh = plsc.VectorSubcoreMesh(core_axis_name="c", subcore_axis_name="s")

    @functools.partial(
        pl.kernel, mesh=mesh,
        out_type=jax.ShapeDtypeStruct((s, DW), jnp.float32),
        scratch_types=[
            pltpu.VMEM((chunks, CH), jnp.int32),
            pltpu.VMEM((CH, DW), jnp.float32),
            pltpu.VMEM((CH, DW), jnp.float32),
            pltpu.SemaphoreType.DMA,
            pltpu.SemaphoreType.DMA,
        ],
    )
    def combine_kernel(dest_hbm, ys_hbm, out_hbm,
                       dst_v, buf0, buf1, sem_g, sem_s):
        wid = lax.axis_index("s") * NC + lax.axis_index("c")
        r0 = wid * chunks * CH
        pltpu.sync_copy(dest_hbm.at[wid], dst_v)
        bufs = (buf0, buf1)
        gets = {}
        puts = {}
        gets[0] = pltpu.async_copy(ys_hbm.at[dst_v.at[0]], bufs[0], sem_g)
        for j in range(chunks):
            b = bufs[j % 2]
            gets[j].wait()
            if j >= 1:
                puts[j - 1].wait()
            if j + 1 < chunks:
                gets[j + 1] = pltpu.async_copy(
                    ys_hbm.at[dst_v.at[j + 1]], bufs[(j + 1) % 2], sem_g)
            puts[j] = pltpu.async_copy(
                b, out_hbm.at[pl.ds(r0 + j * CH, CH)], sem_s)
        puts[chunks - 1].wait()

    return combine_kernel


def _ustrict():
    i = np.arange(SB, dtype=np.int32)
    return jnp.asarray((i[:, None] < i[None, :]).astype(np.float32))


@jax.jit
def kernel(hidden_states, gate_weight, expert_gate_w, expert_up_w,
           expert_down_w, shared_gate_w, shared_up_w, shared_down_w):
    orig_shape = hidden_states.shape
    x = hidden_states.reshape(-1, D)
    t = x.shape[0]
    s = t * TOP_K
    s_pad = s + E * BG
    nblk = s // SB
    tb = min(t, 512)

    topk_e, topk_w = pl.pallas_call(
        _gate_kernel,
        grid=(t // tb,),
        in_specs=[
            pl.BlockSpec((tb, D), lambda i: (i, 0)),
            pl.BlockSpec((E, D), lambda i: (0, 0)),
        ],
        out_specs=[
            pl.BlockSpec((tb, TOP_K), lambda i: (i, 0)),
            pl.BlockSpec((tb, TOP_K), lambda i: (i, 0)),
        ],
        out_shape=[
            jax.ShapeDtypeStruct((t, TOP_K), jnp.int32),
            jax.ShapeDtypeStruct((t, TOP_K), jnp.float32),
        ],
    )(x, gate_weight)

    eflat3d = topk_e.reshape(nblk, 1, SB)
    dest3d, gid128 = pl.pallas_call(
        _sort_kernel,
        grid=(2, nblk),
        in_specs=[
            pl.BlockSpec((1, 1, SB), lambda p, b: (b, 0, 0)),
            pl.BlockSpec((SB, SB), lambda p, b: (0, 0)),
        ],
        out_specs=[
            pl.BlockSpec((1, 1, SB), lambda p, b: (b, 0, 0)),
            pl.BlockSpec((1, 128), lambda p, b: (0, 0)),
        ],
        out_shape=[
            jax.ShapeDtypeStruct((nblk, 1, SB), jnp.int32),
            jax.ShapeDtypeStruct((1, 128), jnp.int32),
        ],
        scratch_shapes=[
            pltpu.VMEM((E, 128), jnp.float32),
            pltpu.VMEM((E, 128), jnp.float32),
            pltpu.VMEM((E, 128), jnp.float32),
        ],
    )(eflat3d, _ustrict())
    gid = gid128.reshape(128)[:s_pad // BG]

    bf = jnp.bfloat16
    tokidx = jnp.asarray((np.arange(s, dtype=np.int32) // TOP_K).reshape(
        NW, s // CH // NW, CH))
    dest_t = dest3d.reshape(NW, s // CH // NW, CH)

    xs = _make_dispatch_kernel(s, s_pad, t)(tokidx, dest_t, x)

    ys = pl.pallas_call(
        _group_mm_kernel,
        grid_spec=pltpu.PrefetchScalarGridSpec(
            num_scalar_prefetch=1,
            grid=(s_pad // BG,),
            in_specs=[
                pl.BlockSpec((BG, D), lambda g, gid_r: (g, 0)),
                pl.BlockSpec((1, D, F), lambda g, gid_r: (gid_r[g], 0, 0)),
                pl.BlockSpec((1, D, F), lambda g, gid_r: (gid_r[g], 0, 0)),
                pl.BlockSpec((1, F, D), lambda g, gid_r: (gid_r[g], 0, 0)),
            ],
            out_specs=pl.BlockSpec((BG, D), lambda g, gid_r: (g, 0)),
        ),
        out_shape=jax.ShapeDtypeStruct((s_pad, D), jnp.float32),
        compiler_params=pltpu.CompilerParams(
            dimension_semantics=("arbitrary",)),
    )(gid, xs, expert_gate_w.astype(bf), expert_up_w.astype(bf),
      expert_down_w.astype(bf))

    ygf = _make_combine_kernel(s, s_pad)(dest_t, ys)
    yg = ygf.reshape(t, TOP_K * D)

    tb2 = min(t, 256)
    y = pl.pallas_call(
        _final_kernel,
        grid=(t // tb2,),
        in_specs=[
            pl.BlockSpec((tb2, D), lambda i: (i, 0)),
            pl.BlockSpec((tb2, TOP_K * D), lambda i: (i, 0)),
            pl.BlockSpec((tb2, TOP_K), lambda i: (i, 0)),
            pl.BlockSpec((D, TOP_K * F), lambda i: (0, 0)),
            pl.BlockSpec((D, TOP_K * F), lambda i: (0, 0)),
            pl.BlockSpec((TOP_K * F, D), lambda i: (0, 0)),
        ],
        out_specs=pl.BlockSpec((tb2, D), lambda i: (i, 0)),
        out_shape=jax.ShapeDtypeStruct((t, D), jnp.float32),
    )(x, yg, topk_w, shared_gate_w.astype(bf), shared_up_w.astype(bf),
      shared_down_w.astype(bf))

    return y.reshape(orig_shape)
